# baseline (speedup 1.0000x reference)
import jax
import jax.numpy as jnp
from jax.experimental import pallas as pl
from jax.experimental.pallas import tpu as pltpu

_B, _VOC, _D, _K = 4096, 1000, 16, 20
_BB = 128
_NR = _BB * _K


def _log_sigmoid(x):
    return jnp.minimum(x, 0.0) - jnp.log1p(jnp.exp(-jnp.abs(x)))


def _body(vi_ref, vo_ref, neg_ref, V_ref, U_ref, P_ref, out_ref):
    V = V_ref[...]
    U = U_ref[...]
    vi_e = jnp.dot(vi_ref[...], V, preferred_element_type=jnp.float32)
    vo_e = jnp.dot(vo_ref[...], U, preferred_element_type=jnp.float32)
    left = _log_sigmoid(jnp.sum(vi_e * vo_e, axis=1, keepdims=True))

    ne = jnp.dot(neg_ref[...], U, preferred_element_type=jnp.float32)
    vi_rep = jnp.dot(P_ref[...], vi_e, preferred_element_type=jnp.float32)
    bm = jnp.sum(ne * vi_rep, axis=1, keepdims=True)
    right = _log_sigmoid(-bm)

    partial = -(jnp.sum(left) + jnp.sum(right)) * (1.0 / _B)

    @pl.when(pl.program_id(0) == 0)
    def _():
        out_ref[0, 0] = 0.0

    out_ref[0, 0] += partial


def kernel(vi, vo, neg_samples, V, U):
    neg2 = neg_samples.reshape(_B * _K, _VOC)
    P = jnp.repeat(jnp.eye(_BB, dtype=jnp.float32), _K, axis=0)
    out = pl.pallas_call(
        _body,
        grid=(_B // _BB,),
        in_specs=[
            pl.BlockSpec((_BB, _VOC), lambda i: (i, 0)),
            pl.BlockSpec((_BB, _VOC), lambda i: (i, 0)),
            pl.BlockSpec((_NR, _VOC), lambda i: (i, 0)),
            pl.BlockSpec((_VOC, _D), lambda i: (0, 0)),
            pl.BlockSpec((_VOC, _D), lambda i: (0, 0)),
            pl.BlockSpec((_NR, _BB), lambda i: (0, 0)),
        ],
        out_specs=pl.BlockSpec(memory_space=pltpu.SMEM),
        out_shape=jax.ShapeDtypeStruct((1, 1), jnp.float32),
    )(vi, vo, neg2, V, U, P)
    return out[0, 0]


# speedup vs baseline: 1.5344x; 1.5344x over previous
import jax
import jax.numpy as jnp
from jax.experimental import pallas as pl
from jax.experimental.pallas import tpu as pltpu

_B, _VOC, _D, _K = 4096, 1000, 16, 20
_BB = 128
_NR = _BB * _K


def _log_sigmoid(x):
    return jnp.minimum(x, 0.0) - jnp.log1p(jnp.exp(-jnp.abs(x)))


def _neg_copy(neg_hbm, buf_ref, sem, step, slot, k):
    return pltpu.make_async_copy(
        neg_hbm.at[pl.ds(step * _BB, _BB), k, :],
        buf_ref.at[slot, pl.ds(k * _BB, _BB), :],
        sem.at[slot],
    )


def _body(vi_ref, vo_ref, neg_hbm, V_ref, U_ref, P_ref, out_ref, buf_ref, sem):
    i = pl.program_id(0)
    n = pl.num_programs(0)
    slot = jax.lax.rem(i, 2)

    @pl.when(i == 0)
    def _():
        for k in range(_K):
            _neg_copy(neg_hbm, buf_ref, sem, i, slot, k).start()

    @pl.when(i + 1 < n)
    def _():
        for k in range(_K):
            _neg_copy(neg_hbm, buf_ref, sem, i + 1, jax.lax.rem(i + 1, 2), k).start()

    for k in range(_K):
        _neg_copy(neg_hbm, buf_ref, sem, i, slot, k).wait()

    V = V_ref[...]
    U = U_ref[...]
    vi_e = jnp.dot(vi_ref[...], V, preferred_element_type=jnp.float32)
    vo_e = jnp.dot(vo_ref[...], U, preferred_element_type=jnp.float32)
    left = _log_sigmoid(jnp.sum(vi_e * vo_e, axis=1, keepdims=True))

    ne = jnp.dot(buf_ref[slot], U, preferred_element_type=jnp.float32)
    vi_rep = jnp.dot(P_ref[...], vi_e, preferred_element_type=jnp.float32)
    bm = jnp.sum(ne * vi_rep, axis=1, keepdims=True)
    right = _log_sigmoid(-bm)

    partial = -(jnp.sum(left) + jnp.sum(right)) * (1.0 / _B)

    @pl.when(i == 0)
    def _():
        out_ref[0, 0] = 0.0

    out_ref[0, 0] += partial


def kernel(vi, vo, neg_samples, V, U):
    P = jnp.tile(jnp.eye(_BB, dtype=jnp.float32), (_K, 1))
    out = pl.pallas_call(
        _body,
        grid=(_B // _BB,),
        in_specs=[
            pl.BlockSpec((_BB, _VOC), lambda i: (i, 0)),
            pl.BlockSpec((_BB, _VOC), lambda i: (i, 0)),
            pl.BlockSpec(memory_space=pl.ANY),
            pl.BlockSpec((_VOC, _D), lambda i: (0, 0)),
            pl.BlockSpec((_VOC, _D), lambda i: (0, 0)),
            pl.BlockSpec((_NR, _BB), lambda i: (0, 0)),
        ],
        out_specs=pl.BlockSpec(memory_space=pltpu.SMEM),
        out_shape=jax.ShapeDtypeStruct((1, 1), jnp.float32),
        scratch_shapes=[
            pltpu.VMEM((2, _NR, _VOC), jnp.float32),
            pltpu.SemaphoreType.DMA((2,)),
        ],
    )(vi, vo, neg_samples, V, U, P)
    return out[0, 0]


# speedup vs baseline: 6.7927x; 4.4268x over previous
import jax
import jax.numpy as jnp
from jax.experimental import pallas as pl
from jax.experimental.pallas import tpu as pltpu

_B, _VOC, _D, _K = 4096, 1000, 16, 20
_BB = 128


def _log_sigmoid(x):
    return jnp.minimum(x, 0.0) - jnp.log1p(jnp.exp(-jnp.abs(x)))


def _body(viT_ref, voT_ref, negT_ref, VT_ref, UT_ref, out_ref):
    VT = VT_ref[...]
    UT = UT_ref[...]
    vi_eT = jnp.dot(VT, viT_ref[...], preferred_element_type=jnp.float32)
    vo_eT = jnp.dot(UT, voT_ref[...], preferred_element_type=jnp.float32)
    acc = _log_sigmoid(jnp.sum(vi_eT * vo_eT, axis=0, keepdims=True))
    for k in range(_K):
        neT = jnp.dot(UT, negT_ref[k], preferred_element_type=jnp.float32)
        bm_k = jnp.sum(neT * vi_eT, axis=0, keepdims=True)
        acc = acc + _log_sigmoid(-bm_k)
    partial = -jnp.sum(acc) * (1.0 / _B)

    @pl.when(pl.program_id(0) == 0)
    def _():
        out_ref[0, 0] = 0.0

    out_ref[0, 0] += partial


def kernel(vi, vo, neg_samples, V, U):
    viT = vi.T
    voT = vo.T
    negT = jnp.transpose(neg_samples, (1, 2, 0))
    VT = V.T
    UT = U.T
    out = pl.pallas_call(
        _body,
        grid=(_B // _BB,),
        in_specs=[
            pl.BlockSpec((_VOC, _BB), lambda i: (0, i)),
            pl.BlockSpec((_VOC, _BB), lambda i: (0, i)),
            pl.BlockSpec((_K, _VOC, _BB), lambda i: (0, 0, i)),
            pl.BlockSpec((_D, _VOC), lambda i: (0, 0)),
            pl.BlockSpec((_D, _VOC), lambda i: (0, 0)),
        ],
        out_specs=pl.BlockSpec(memory_space=pltpu.SMEM),
        out_shape=jax.ShapeDtypeStruct((1, 1), jnp.float32),
    )(viT, voT, negT, VT, UT)
    return out[0, 0]


# speedup vs baseline: 6.9085x; 1.0170x over previous
import jax
import jax.numpy as jnp
from jax.experimental import pallas as pl
from jax.experimental.pallas import tpu as pltpu

_B, _VOC, _D, _K = 4096, 1000, 16, 20
_BB = 256


def _log_sigmoid(x):
    return jnp.minimum(x, 0.0) - jnp.log1p(jnp.exp(-jnp.abs(x)))


def _body(viT_ref, voT_ref, negT_ref, VT_ref, UT_ref, out_ref):
    VT = VT_ref[...]
    UT = UT_ref[...]
    vi_eT = jnp.dot(VT, viT_ref[...], preferred_element_type=jnp.float32)
    vo_eT = jnp.dot(UT, voT_ref[...], preferred_element_type=jnp.float32)
    acc = _log_sigmoid(jnp.sum(vi_eT * vo_eT, axis=0, keepdims=True))
    for k in range(_K):
        neT = jnp.dot(UT, negT_ref[k], preferred_element_type=jnp.float32)
        bm_k = jnp.sum(neT * vi_eT, axis=0, keepdims=True)
        acc = acc + _log_sigmoid(-bm_k)
    partial = -jnp.sum(acc) * (1.0 / _B)

    @pl.when(pl.program_id(0) == 0)
    def _():
        out_ref[0, 0] = 0.0

    out_ref[0, 0] += partial


def kernel(vi, vo, neg_samples, V, U):
    viT = vi.T
    voT = vo.T
    negT = jnp.transpose(neg_samples, (1, 2, 0))
    VT = V.T
    UT = U.T
    out = pl.pallas_call(
        _body,
        grid=(_B // _BB,),
        in_specs=[
            pl.BlockSpec((_VOC, _BB), lambda i: (0, i)),
            pl.BlockSpec((_VOC, _BB), lambda i: (0, i)),
            pl.BlockSpec((_K, _VOC, _BB), lambda i: (0, 0, i)),
            pl.BlockSpec((_D, _VOC), lambda i: (0, 0)),
            pl.BlockSpec((_D, _VOC), lambda i: (0, 0)),
        ],
        out_specs=pl.BlockSpec(memory_space=pltpu.SMEM),
        out_shape=jax.ShapeDtypeStruct((1, 1), jnp.float32),
    )(viT, voT, negT, VT, UT)
    return out[0, 0]


# speedup vs baseline: 6.9827x; 1.0108x over previous
import jax
import jax.numpy as jnp
from jax.experimental import pallas as pl
from jax.experimental.pallas import tpu as pltpu

_B, _VOC, _D, _K = 4096, 1000, 16, 20
_BB = 256


def _log_sigmoid(x):
    return jnp.minimum(x, 0.0) - jnp.log1p(jnp.exp(-jnp.abs(x)))


def _body(viT_ref, voT_ref, negA_ref, negB_ref, VT_ref, UT_ref, out_ref):
    VT = VT_ref[...]
    UT = UT_ref[...]
    vi_eT = jnp.dot(VT, viT_ref[...], preferred_element_type=jnp.float32)
    vo_eT = jnp.dot(UT, voT_ref[...], preferred_element_type=jnp.float32)
    acc = _log_sigmoid(jnp.sum(vi_eT * vo_eT, axis=0, keepdims=True))
    for negT_ref in (negA_ref, negB_ref):
        for k in range(_K // 2):
            neT = jnp.dot(UT, negT_ref[k], preferred_element_type=jnp.float32)
            bm_k = jnp.sum(neT * vi_eT, axis=0, keepdims=True)
            acc = acc + _log_sigmoid(-bm_k)
    partial = -jnp.sum(acc) * (1.0 / _B)

    @pl.when(pl.program_id(0) == 0)
    def _():
        out_ref[0, 0] = 0.0

    out_ref[0, 0] += partial


def kernel(vi, vo, neg_samples, V, U):
    viT = vi.T
    voT = vo.T
    negT = jnp.transpose(neg_samples, (1, 2, 0))
    VT = V.T
    UT = U.T
    out = pl.pallas_call(
        _body,
        grid=(_B // _BB,),
        in_specs=[
            pl.BlockSpec((_VOC, _BB), lambda i: (0, i)),
            pl.BlockSpec((_VOC, _BB), lambda i: (0, i)),
            pl.BlockSpec((_K // 2, _VOC, _BB), lambda i: (0, 0, i)),
            pl.BlockSpec((_K // 2, _VOC, _BB), lambda i: (1, 0, i)),
            pl.BlockSpec((_D, _VOC), lambda i: (0, 0)),
            pl.BlockSpec((_D, _VOC), lambda i: (0, 0)),
        ],
        out_specs=pl.BlockSpec(memory_space=pltpu.SMEM),
        out_shape=jax.ShapeDtypeStruct((1, 1), jnp.float32),
    )(viT, voT, negT, negT, VT, UT)
    return out[0, 0]
